# baseline (speedup 1.0000x reference)
import jax
import jax.numpy as jnp
from jax import lax
from jax.experimental import pallas as pl
from jax.experimental.pallas import tpu as pltpu
from jax.experimental.pallas import tpu_sc as plsc

N = 10000
D = 128
C = 10
NC = 2
NS = 16
NW = NC * NS
K = 128
CH = 80
EW = CH * K
EP = NW * EW
NP = 10112
RPT = NP // NS

_MESH = plsc.VectorSubcoreMesh(core_axis_name="c", subcore_axis_name="s")


G = 4
TCH = EP // K
CH0 = 32
CH1 = (TCH - NS * CH0) // NS
assert CH0 % (2 * G) == 0 and CH1 % (2 * G) == 0 and NS * (CH0 + CH1) == TCH


def _agg_body(h_hbm, e_hbm, zacc_hbm, acc_out,
              idx0, idx1, rows0, rows1, acc_sh, sg0, sg1, si0, si1):
    cid = lax.axis_index("c")
    sid = lax.axis_index("s")
    idxb = (idx0, idx1)
    rows = (rows0, rows1)
    sg = (sg0, sg1)
    si = (si0, si1)
    ch_w = lax.select(cid == 0, jnp.int32(CH0), jnp.int32(CH1))
    b0 = cid * (NS * CH0) + sid * ch_w
    ng = ch_w // G
    r0 = sid * RPT
    pltpu.sync_copy(zacc_hbm.at[pl.ds(r0, RPT)], acc_sh.at[pl.ds(r0, RPT)])
    pltpu.sync_copy(e_hbm.at[pl.ds(b0, G)], idx0)
    pltpu.async_copy(h_hbm.at[idx0.at[0, 0]], rows0, sg0)
    plsc.subcore_barrier()

    def wait_gather(b):
        pltpu.make_async_copy(h_hbm.at[pl.ds(0, K)], rows[b], sg[b]).wait()

    def wait_idx(q):
        pltpu.make_async_copy(e_hbm.at[pl.ds(0, G)], idxb[q],
                              si[q]).wait()

    def super_group(sg_i, c):
        for q in (0, 1):
            gidx = sg_i * 2 + q
            @pl.when(gidx + 1 < ng)
            def _():
                pltpu.async_copy(e_hbm.at[pl.ds(b0 + (gidx + 1) * G, G)],
                                 idxb[1 - q], si[1 - q])
            for p in range(G):
                b = p % 2
                j = gidx * G + p
                if p < G - 1:
                    pltpu.async_copy(h_hbm.at[idxb[q].at[p + 1, 0]],
                                     rows[1 - b], sg[1 - b])
                else:
                    @pl.when(j + 1 < ch_w)
                    def _():
                        wait_idx(1 - q)
                        pltpu.async_copy(h_hbm.at[idxb[1 - q].at[0, 0]],
                                         rows[1 - b], sg[1 - b])
                wait_gather(b)
                pltpu.sync_copy(rows[b], acc_sh.at[idxb[q].at[p, 1]],
                                add=True)
        return c

    lax.fori_loop(0, ng // 2, super_group, 0)
    plsc.subcore_barrier()
    pltpu.sync_copy(acc_sh.at[pl.ds(r0, RPT)],
                    acc_out.at[cid, pl.ds(r0, RPT)])


_agg = pl.kernel(
    _agg_body,
    out_type=[jax.ShapeDtypeStruct((NC, NP, D), jnp.float32)],
    mesh=_MESH,
    scratch_types=[
        pltpu.VMEM((G, 2, K), jnp.int32),
        pltpu.VMEM((G, 2, K), jnp.int32),
        pltpu.VMEM((K, D), jnp.float32),
        pltpu.VMEM((K, D), jnp.float32),
        pltpu.VMEM_SHARED((NP, D), jnp.float32),
        pltpu.SemaphoreType.DMA,
        pltpu.SemaphoreType.DMA,
        pltpu.SemaphoreType.DMA,
        pltpu.SemaphoreType.DMA,
    ],
)


def _deg_body(dst_hbm, zdeg_hbm, ones_hbm, deg_out, idx_d, ones_v, deg_sh):
    cid = lax.axis_index("c")
    sid = lax.axis_index("s")
    wid = cid * NS + sid
    pltpu.sync_copy(dst_hbm.at[wid], idx_d)
    pltpu.sync_copy(ones_hbm, ones_v)
    r0 = sid * RPT
    pltpu.sync_copy(zdeg_hbm.at[pl.ds(r0, RPT)], deg_sh.at[pl.ds(r0, RPT)])
    plsc.subcore_barrier()

    def step(j, c):
        pltpu.sync_copy(ones_v, deg_sh.at[idx_d.at[j]], add=True)
        return c

    lax.fori_loop(0, CH, step, 0)
    plsc.subcore_barrier()
    pltpu.sync_copy(deg_sh.at[pl.ds(r0, RPT)],
                    deg_out.at[cid, pl.ds(r0, RPT)])


_deg = pl.kernel(
    _deg_body,
    out_type=[jax.ShapeDtypeStruct((NC, NP, D), jnp.float32)],
    mesh=_MESH,
    scratch_types=[
        pltpu.VMEM((CH, K), jnp.int32),
        pltpu.VMEM((K, D), jnp.float32),
        pltpu.VMEM_SHARED((NP, D), jnp.float32),
    ],
)

_BLK = 1000


def _deg_col(d_ref):
    return d_ref[0, :, :1] + d_ref[1, :, :1]


def _tc1_body(x_ref, s0_ref, s1_ref, d_ref, ws_ref, wn_ref, b_ref, o_ref):
    deg = _deg_col(d_ref)
    hn = (s0_ref[...] + s1_ref[...]) / jnp.maximum(deg, 1.0)
    o_ref[...] = jnp.maximum(
        jnp.dot(x_ref[...], ws_ref[...], preferred_element_type=jnp.float32)
        + jnp.dot(hn, wn_ref[...], preferred_element_type=jnp.float32)
        + b_ref[...], 0.0)


def _tc1(x, s0, s1, dp, ws, wn, b):
    grid = N // _BLK
    row = lambda i: (i, 0)
    full = lambda i: (0, 0)
    return pl.pallas_call(
        _tc1_body,
        grid=(grid,),
        in_specs=[
            pl.BlockSpec((_BLK, D), row),
            pl.BlockSpec((_BLK, D), row),
            pl.BlockSpec((_BLK, D), row),
            pl.BlockSpec((NC, _BLK, D), lambda i: (0, i, 0)),
            pl.BlockSpec((D, D), full),
            pl.BlockSpec((D, D), full),
            pl.BlockSpec((1, D), full),
        ],
        out_specs=pl.BlockSpec((_BLK, D), row),
        out_shape=jax.ShapeDtypeStruct((N, D), jnp.float32),
    )(x, s0, s1, dp, ws, wn, b)


def _tc2_body(h_ref, s0_ref, s1_ref, d_ref, ws_ref, wn_ref, b_ref,
              wc_ref, bc_ref, o_ref, acc_ref):
    i = pl.program_id(0)

    @pl.when(i == 0)
    def _():
        acc_ref[...] = jnp.zeros_like(acc_ref)

    deg = _deg_col(d_ref)
    hn = (s0_ref[...] + s1_ref[...]) / jnp.maximum(deg, 1.0)
    h2 = jnp.maximum(
        jnp.dot(h_ref[...], ws_ref[...], preferred_element_type=jnp.float32)
        + jnp.dot(hn, wn_ref[...], preferred_element_type=jnp.float32)
        + b_ref[...], 0.0)
    acc_ref[...] += jnp.sum(h2, axis=0, keepdims=True)

    @pl.when(i == pl.num_programs(0) - 1)
    def _():
        pooled = acc_ref[...] * (1.0 / N)
        o_ref[...] = (jnp.dot(pooled, wc_ref[...],
                              preferred_element_type=jnp.float32)
                      + bc_ref[...])


def _tc2(h, s0, s1, dp, ws, wn, b, wc, bc):
    grid = N // _BLK
    row = lambda i: (i, 0)
    full = lambda i: (0, 0)
    out = pl.pallas_call(
        _tc2_body,
        grid=(grid,),
        in_specs=[
            pl.BlockSpec((_BLK, D), row),
            pl.BlockSpec((_BLK, D), row),
            pl.BlockSpec((_BLK, D), row),
            pl.BlockSpec((NC, _BLK, D), lambda i: (0, i, 0)),
            pl.BlockSpec((D, D), full),
            pl.BlockSpec((D, D), full),
            pl.BlockSpec((1, D), full),
            pl.BlockSpec((D, C), full),
            pl.BlockSpec((1, C), full),
        ],
        out_specs=pl.BlockSpec((1, C), full),
        out_shape=jax.ShapeDtypeStruct((1, C), jnp.float32),
        scratch_shapes=[pltpu.VMEM((1, D), jnp.float32)],
    )(h, s0, s1, dp, ws, wn, b, wc, bc)
    return out[0]


def kernel(x, edge_index, W_self0, W_neigh0, b0, W_self1, W_neigh1, b1,
           W_cls, b_cls):
    src = edge_index[0]
    dst = edge_index[1]
    pad = EP - src.shape[0]
    srcp = jnp.concatenate([src, jnp.zeros((pad,), jnp.int32)])
    dst_pad = N + (jnp.arange(pad, dtype=jnp.int32) % (NP - N))
    dstp = jnp.concatenate([dst, dst_pad])
    e3 = jnp.stack([srcp.reshape(TCH, K), dstp.reshape(TCH, K)], axis=1)
    dst3 = dstp.reshape(NW, CH, K)

    zacc = jnp.zeros((NP, D), jnp.float32)
    ones128 = jnp.ones((K, D), jnp.float32)

    degp, = _deg(dst3, zacc, ones128)
    acc1, = _agg(x, e3, zacc)
    h1 = _tc1(x, acc1[0], acc1[1], degp, W_self0, W_neigh0, b0.reshape(1, D))
    acc2, = _agg(h1, e3, zacc)
    return _tc2(h1, acc2[0], acc2[1], degp, W_self1, W_neigh1,
                b1.reshape(1, D), W_cls, b_cls.reshape(1, C))


# speedup vs baseline: 1.1107x; 1.1107x over previous
import jax
import jax.numpy as jnp
from jax import lax
from jax.experimental import pallas as pl
from jax.experimental.pallas import tpu as pltpu
from jax.experimental.pallas import tpu_sc as plsc

N = 10000
D = 128
C = 10
NC = 2
NS = 16
NW = NC * NS
K = 128
CH = 80
EW = CH * K
EP = NW * EW
NP = 10112
RPT = NP // NS

_MESH = plsc.VectorSubcoreMesh(core_axis_name="c", subcore_axis_name="s")


G = 4
TCH = EP // K
CH0 = 112
CH1 = (TCH - NS * CH0) // NS
assert CH0 % (2 * G) == 0 and CH1 % (2 * G) == 0 and NS * (CH0 + CH1) == TCH


def _agg_body(h_hbm, e_hbm, zacc_hbm, acc_out,
              idx0, idx1, rows0, rows1, acc_sh, sg0, sg1, si0, si1):
    cid = lax.axis_index("c")
    sid = lax.axis_index("s")
    idxb = (idx0, idx1)
    rows = (rows0, rows1)
    sg = (sg0, sg1)
    si = (si0, si1)
    ch_w = lax.select(cid == 0, jnp.int32(CH0), jnp.int32(CH1))
    b0 = cid * (NS * CH0) + sid * ch_w
    ng = ch_w // G
    r0 = sid * RPT
    pltpu.sync_copy(zacc_hbm.at[pl.ds(r0, RPT)], acc_sh.at[pl.ds(r0, RPT)])
    pltpu.sync_copy(e_hbm.at[pl.ds(b0, G)], idx0)
    pltpu.async_copy(h_hbm.at[idx0.at[0, 0]], rows0, sg0)
    plsc.subcore_barrier()

    def wait_gather(b):
        pltpu.make_async_copy(h_hbm.at[pl.ds(0, K)], rows[b], sg[b]).wait()

    def wait_idx(q):
        pltpu.make_async_copy(e_hbm.at[pl.ds(0, G)], idxb[q],
                              si[q]).wait()

    def super_group(sg_i, c):
        for q in (0, 1):
            gidx = sg_i * 2 + q
            @pl.when(gidx + 1 < ng)
            def _():
                pltpu.async_copy(e_hbm.at[pl.ds(b0 + (gidx + 1) * G, G)],
                                 idxb[1 - q], si[1 - q])
            for p in range(G):
                b = p % 2
                j = gidx * G + p
                if p < G - 1:
                    pltpu.async_copy(h_hbm.at[idxb[q].at[p + 1, 0]],
                                     rows[1 - b], sg[1 - b])
                else:
                    @pl.when(j + 1 < ch_w)
                    def _():
                        wait_idx(1 - q)
                        pltpu.async_copy(h_hbm.at[idxb[1 - q].at[0, 0]],
                                         rows[1 - b], sg[1 - b])
                wait_gather(b)
                pltpu.sync_copy(rows[b], acc_sh.at[idxb[q].at[p, 1]],
                                add=True)
        return c

    lax.fori_loop(0, ng // 2, super_group, 0)
    plsc.subcore_barrier()
    pltpu.sync_copy(acc_sh.at[pl.ds(r0, RPT)],
                    acc_out.at[cid, pl.ds(r0, RPT)])


_agg = pl.kernel(
    _agg_body,
    out_type=[jax.ShapeDtypeStruct((NC, NP, D), jnp.float32)],
    mesh=_MESH,
    scratch_types=[
        pltpu.VMEM((G, 2, K), jnp.int32),
        pltpu.VMEM((G, 2, K), jnp.int32),
        pltpu.VMEM((K, D), jnp.float32),
        pltpu.VMEM((K, D), jnp.float32),
        pltpu.VMEM_SHARED((NP, D), jnp.float32),
        pltpu.SemaphoreType.DMA,
        pltpu.SemaphoreType.DMA,
        pltpu.SemaphoreType.DMA,
        pltpu.SemaphoreType.DMA,
    ],
)


def _deg_body(dst_hbm, zdeg_hbm, ones_hbm, deg_out, idx_d, ones_v, deg_sh):
    cid = lax.axis_index("c")
    sid = lax.axis_index("s")
    wid = cid * NS + sid
    pltpu.sync_copy(dst_hbm.at[wid], idx_d)
    pltpu.sync_copy(ones_hbm, ones_v)
    r0 = sid * RPT
    pltpu.sync_copy(zdeg_hbm.at[pl.ds(r0, RPT)], deg_sh.at[pl.ds(r0, RPT)])
    plsc.subcore_barrier()

    def step(j, c):
        pltpu.sync_copy(ones_v, deg_sh.at[idx_d.at[j]], add=True)
        return c

    lax.fori_loop(0, CH, step, 0)
    plsc.subcore_barrier()
    pltpu.sync_copy(deg_sh.at[pl.ds(r0, RPT)],
                    deg_out.at[cid, pl.ds(r0, RPT)])


_deg = pl.kernel(
    _deg_body,
    out_type=[jax.ShapeDtypeStruct((NC, NP, D), jnp.float32)],
    mesh=_MESH,
    scratch_types=[
        pltpu.VMEM((CH, K), jnp.int32),
        pltpu.VMEM((K, D), jnp.float32),
        pltpu.VMEM_SHARED((NP, D), jnp.float32),
    ],
)

_BLK = 1000


def _deg_col(d_ref):
    return d_ref[0, :, :1] + d_ref[1, :, :1]


def _tc1_body(x_ref, s0_ref, s1_ref, d_ref, ws_ref, wn_ref, b_ref, o_ref):
    deg = _deg_col(d_ref)
    hn = (s0_ref[...] + s1_ref[...]) / jnp.maximum(deg, 1.0)
    o_ref[...] = jnp.maximum(
        jnp.dot(x_ref[...], ws_ref[...], preferred_element_type=jnp.float32)
        + jnp.dot(hn, wn_ref[...], preferred_element_type=jnp.float32)
        + b_ref[...], 0.0)


def _tc1(x, s0, s1, dp, ws, wn, b):
    grid = N // _BLK
    row = lambda i: (i, 0)
    full = lambda i: (0, 0)
    return pl.pallas_call(
        _tc1_body,
        grid=(grid,),
        in_specs=[
            pl.BlockSpec((_BLK, D), row),
            pl.BlockSpec((_BLK, D), row),
            pl.BlockSpec((_BLK, D), row),
            pl.BlockSpec((NC, _BLK, D), lambda i: (0, i, 0)),
            pl.BlockSpec((D, D), full),
            pl.BlockSpec((D, D), full),
            pl.BlockSpec((1, D), full),
        ],
        out_specs=pl.BlockSpec((_BLK, D), row),
        out_shape=jax.ShapeDtypeStruct((N, D), jnp.float32),
    )(x, s0, s1, dp, ws, wn, b)


def _tc2_body(h_ref, s0_ref, s1_ref, d_ref, ws_ref, wn_ref, b_ref,
              wc_ref, bc_ref, o_ref, acc_ref):
    i = pl.program_id(0)

    @pl.when(i == 0)
    def _():
        acc_ref[...] = jnp.zeros_like(acc_ref)

    deg = _deg_col(d_ref)
    hn = (s0_ref[...] + s1_ref[...]) / jnp.maximum(deg, 1.0)
    h2 = jnp.maximum(
        jnp.dot(h_ref[...], ws_ref[...], preferred_element_type=jnp.float32)
        + jnp.dot(hn, wn_ref[...], preferred_element_type=jnp.float32)
        + b_ref[...], 0.0)
    acc_ref[...] += jnp.sum(h2, axis=0, keepdims=True)

    @pl.when(i == pl.num_programs(0) - 1)
    def _():
        pooled = acc_ref[...] * (1.0 / N)
        o_ref[...] = (jnp.dot(pooled, wc_ref[...],
                              preferred_element_type=jnp.float32)
                      + bc_ref[...])


def _tc2(h, s0, s1, dp, ws, wn, b, wc, bc):
    grid = N // _BLK
    row = lambda i: (i, 0)
    full = lambda i: (0, 0)
    out = pl.pallas_call(
        _tc2_body,
        grid=(grid,),
        in_specs=[
            pl.BlockSpec((_BLK, D), row),
            pl.BlockSpec((_BLK, D), row),
            pl.BlockSpec((_BLK, D), row),
            pl.BlockSpec((NC, _BLK, D), lambda i: (0, i, 0)),
            pl.BlockSpec((D, D), full),
            pl.BlockSpec((D, D), full),
            pl.BlockSpec((1, D), full),
            pl.BlockSpec((D, C), full),
            pl.BlockSpec((1, C), full),
        ],
        out_specs=pl.BlockSpec((1, C), full),
        out_shape=jax.ShapeDtypeStruct((1, C), jnp.float32),
        scratch_shapes=[pltpu.VMEM((1, D), jnp.float32)],
    )(h, s0, s1, dp, ws, wn, b, wc, bc)
    return out[0]


def kernel(x, edge_index, W_self0, W_neigh0, b0, W_self1, W_neigh1, b1,
           W_cls, b_cls):
    src = edge_index[0]
    dst = edge_index[1]
    pad = EP - src.shape[0]
    srcp = jnp.concatenate([src, jnp.zeros((pad,), jnp.int32)])
    dst_pad = N + (jnp.arange(pad, dtype=jnp.int32) % (NP - N))
    dstp = jnp.concatenate([dst, dst_pad])
    e3 = jnp.stack([srcp.reshape(TCH, K), dstp.reshape(TCH, K)], axis=1)
    dst3 = dstp.reshape(NW, CH, K)

    zacc = jnp.zeros((NP, D), jnp.float32)
    ones128 = jnp.ones((K, D), jnp.float32)

    degp, = _deg(dst3, zacc, ones128)
    acc1, = _agg(x, e3, zacc)
    h1 = _tc1(x, acc1[0], acc1[1], degp, W_self0, W_neigh0, b0.reshape(1, D))
    acc2, = _agg(h1, e3, zacc)
    return _tc2(h1, acc2[0], acc2[1], degp, W_self1, W_neigh1,
                b1.reshape(1, D), W_cls, b_cls.reshape(1, C))


# speedup vs baseline: 3.2426x; 2.9195x over previous
import jax
import jax.numpy as jnp
from jax import lax
from jax.experimental import pallas as pl
from jax.experimental.pallas import tpu as pltpu
from jax.experimental.pallas import tpu_sc as plsc

N = 10000
D = 128
C = 10
NC = 2
NS = 16
NW = NC * NS
K = 128
CH = 80
EW = CH * K
EP = NW * EW
NP = 10112
RPT = NP // NS

_MESH = plsc.VectorSubcoreMesh(core_axis_name="c", subcore_axis_name="s")


G = 4
TCH = EP // K
CH0 = 80
CH1 = (TCH - NS * CH0) // NS
assert CH0 % (2 * G) == 0 and CH1 % (2 * G) == 0 and NS * (CH0 + CH1) == TCH


def _agg_body(h_hbm, e_hbm, zacc_hbm, acc_out,
              idx0, idx1, rows0, rows1, acc_sh, sg0, sg1, si0, si1):
    cid = lax.axis_index("c")
    sid = lax.axis_index("s")
    idxb = (idx0, idx1)
    rows = (rows0, rows1)
    sg = (sg0, sg1)
    si = (si0, si1)
    ch_w = lax.select(cid == 0, jnp.int32(CH0), jnp.int32(CH1))
    b0 = cid * (NS * CH0) + sid * ch_w
    ng = ch_w // G
    r0 = sid * RPT
    pltpu.sync_copy(zacc_hbm.at[pl.ds(r0, RPT)], acc_sh.at[pl.ds(r0, RPT)])
    pltpu.sync_copy(e_hbm.at[pl.ds(b0, G)], idx0)
    pltpu.async_copy(h_hbm.at[idx0.at[0, 0]], rows0, sg0)
    plsc.subcore_barrier()

    def wait_gather(b):
        pltpu.make_async_copy(h_hbm.at[pl.ds(0, K)], rows[b], sg[b]).wait()

    def wait_idx(q):
        pltpu.make_async_copy(e_hbm.at[pl.ds(0, G)], idxb[q],
                              si[q]).wait()

    def super_group(sg_i, c):
        for q in (0, 1):
            gidx = sg_i * 2 + q
            @pl.when(gidx + 1 < ng)
            def _():
                pltpu.async_copy(e_hbm.at[pl.ds(b0 + (gidx + 1) * G, G)],
                                 idxb[1 - q], si[1 - q])
            for p in range(G):
                b = p % 2
                j = gidx * G + p
                if p < G - 1:
                    pltpu.async_copy(h_hbm.at[idxb[q].at[p + 1, 0]],
                                     rows[1 - b], sg[1 - b])
                else:
                    @pl.when(j + 1 < ch_w)
                    def _():
                        wait_idx(1 - q)
                        pltpu.async_copy(h_hbm.at[idxb[1 - q].at[0, 0]],
                                         rows[1 - b], sg[1 - b])
                wait_gather(b)
                pltpu.sync_copy(rows[b], acc_sh.at[idxb[q].at[p, 1]],
                                add=True)
        return c

    lax.fori_loop(0, ng // 2, super_group, 0)
    plsc.subcore_barrier()
    pltpu.sync_copy(acc_sh.at[pl.ds(r0, RPT)],
                    acc_out.at[cid, pl.ds(r0, RPT)])


_agg = pl.kernel(
    _agg_body,
    out_type=[jax.ShapeDtypeStruct((NC, NP, D), jnp.float32)],
    mesh=_MESH,
    scratch_types=[
        pltpu.VMEM((G, 2, K), jnp.int32),
        pltpu.VMEM((G, 2, K), jnp.int32),
        pltpu.VMEM((K, D), jnp.float32),
        pltpu.VMEM((K, D), jnp.float32),
        pltpu.VMEM_SHARED((NP, D), jnp.float32),
        pltpu.SemaphoreType.DMA,
        pltpu.SemaphoreType.DMA,
        pltpu.SemaphoreType.DMA,
        pltpu.SemaphoreType.DMA,
    ],
)


def _deg_body(dst_hbm, zdeg_hbm, ones_hbm, deg_out, idx_d, ones_v, deg_sh):
    cid = lax.axis_index("c")
    sid = lax.axis_index("s")
    wid = cid * NS + sid
    pltpu.sync_copy(dst_hbm.at[wid], idx_d)
    pltpu.sync_copy(ones_hbm, ones_v)
    r0 = sid * RPT
    pltpu.sync_copy(zdeg_hbm.at[pl.ds(r0, RPT)], deg_sh.at[pl.ds(r0, RPT)])
    plsc.subcore_barrier()

    def step(j, c):
        pltpu.sync_copy(ones_v, deg_sh.at[idx_d.at[j]], add=True)
        return c

    lax.fori_loop(0, CH, step, 0)
    plsc.subcore_barrier()
    pltpu.sync_copy(deg_sh.at[pl.ds(r0, RPT)],
                    deg_out.at[cid, pl.ds(r0, RPT)])


_deg = pl.kernel(
    _deg_body,
    out_type=[jax.ShapeDtypeStruct((NC, NP, D), jnp.float32)],
    mesh=_MESH,
    scratch_types=[
        pltpu.VMEM((CH, K), jnp.int32),
        pltpu.VMEM((K, D), jnp.float32),
        pltpu.VMEM_SHARED((NP, D), jnp.float32),
    ],
)

_BLK = 1000


def _deg_col(d_ref):
    return d_ref[0, :, :1] + d_ref[1, :, :1]


def _tc1_body(x_ref, s0_ref, s1_ref, d_ref, ws_ref, wn_ref, b_ref, o_ref):
    deg = _deg_col(d_ref)
    hn = (s0_ref[...] + s1_ref[...]) / jnp.maximum(deg, 1.0)
    o_ref[...] = jnp.maximum(
        jnp.dot(x_ref[...], ws_ref[...], preferred_element_type=jnp.float32)
        + jnp.dot(hn, wn_ref[...], preferred_element_type=jnp.float32)
        + b_ref[...], 0.0)


def _tc1(x, s0, s1, dp, ws, wn, b):
    grid = N // _BLK
    row = lambda i: (i, 0)
    full = lambda i: (0, 0)
    return pl.pallas_call(
        _tc1_body,
        grid=(grid,),
        in_specs=[
            pl.BlockSpec((_BLK, D), row),
            pl.BlockSpec((_BLK, D), row),
            pl.BlockSpec((_BLK, D), row),
            pl.BlockSpec((NC, _BLK, D), lambda i: (0, i, 0)),
            pl.BlockSpec((D, D), full),
            pl.BlockSpec((D, D), full),
            pl.BlockSpec((1, D), full),
        ],
        out_specs=pl.BlockSpec((_BLK, D), row),
        out_shape=jax.ShapeDtypeStruct((N, D), jnp.float32),
    )(x, s0, s1, dp, ws, wn, b)


def _tc2_body(h_ref, s0_ref, s1_ref, d_ref, ws_ref, wn_ref, b_ref,
              wc_ref, bc_ref, o_ref, acc_ref):
    i = pl.program_id(0)

    @pl.when(i == 0)
    def _():
        acc_ref[...] = jnp.zeros_like(acc_ref)

    deg = _deg_col(d_ref)
    hn = (s0_ref[...] + s1_ref[...]) / jnp.maximum(deg, 1.0)
    h2 = jnp.maximum(
        jnp.dot(h_ref[...], ws_ref[...], preferred_element_type=jnp.float32)
        + jnp.dot(hn, wn_ref[...], preferred_element_type=jnp.float32)
        + b_ref[...], 0.0)
    acc_ref[...] += jnp.sum(h2, axis=0, keepdims=True)

    @pl.when(i == pl.num_programs(0) - 1)
    def _():
        pooled = acc_ref[...] * (1.0 / N)
        o_ref[...] = (jnp.dot(pooled, wc_ref[...],
                              preferred_element_type=jnp.float32)
                      + bc_ref[...])


def _tc2(h, s0, s1, dp, ws, wn, b, wc, bc):
    grid = N // _BLK
    row = lambda i: (i, 0)
    full = lambda i: (0, 0)
    out = pl.pallas_call(
        _tc2_body,
        grid=(grid,),
        in_specs=[
            pl.BlockSpec((_BLK, D), row),
            pl.BlockSpec((_BLK, D), row),
            pl.BlockSpec((_BLK, D), row),
            pl.BlockSpec((NC, _BLK, D), lambda i: (0, i, 0)),
            pl.BlockSpec((D, D), full),
            pl.BlockSpec((D, D), full),
            pl.BlockSpec((1, D), full),
            pl.BlockSpec((D, C), full),
            pl.BlockSpec((1, C), full),
        ],
        out_specs=pl.BlockSpec((1, C), full),
        out_shape=jax.ShapeDtypeStruct((1, C), jnp.float32),
        scratch_shapes=[pltpu.VMEM((1, D), jnp.float32)],
    )(h, s0, s1, dp, ws, wn, b, wc, bc)
    return out[0]


def kernel(x, edge_index, W_self0, W_neigh0, b0, W_self1, W_neigh1, b1,
           W_cls, b_cls):
    src = edge_index[0]
    dst = edge_index[1]
    pad = EP - src.shape[0]
    src_pad = (jnp.arange(pad, dtype=jnp.int32) * 131) % N
    srcp = jnp.concatenate([src, src_pad])
    dst_pad = N + (jnp.arange(pad, dtype=jnp.int32) % (NP - N))
    dstp = jnp.concatenate([dst, dst_pad])
    e3 = jnp.stack([srcp.reshape(TCH, K), dstp.reshape(TCH, K)], axis=1)
    dst3 = dstp.reshape(NW, CH, K)

    zacc = jnp.zeros((NP, D), jnp.float32)
    ones128 = jnp.ones((K, D), jnp.float32)

    degp, = _deg(dst3, zacc, ones128)
    acc1, = _agg(x, e3, zacc)
    h1 = _tc1(x, acc1[0], acc1[1], degp, W_self0, W_neigh0, b0.reshape(1, D))
    acc2, = _agg(h1, e3, zacc)
    return _tc2(h1, acc2[0], acc2[1], degp, W_self1, W_neigh1,
                b1.reshape(1, D), W_cls, b_cls.reshape(1, C))


# speedup vs baseline: 3.2769x; 1.0106x over previous
import jax
import jax.numpy as jnp
from jax import lax
from jax.experimental import pallas as pl
from jax.experimental.pallas import tpu as pltpu
from jax.experimental.pallas import tpu_sc as plsc

N = 10000
D = 128
C = 10
NC = 2
NS = 16
NW = NC * NS
K = 128
CH = 80
EW = CH * K
EP = NW * EW
NP = 10112
RPT = NP // NS

_MESH = plsc.VectorSubcoreMesh(core_axis_name="c", subcore_axis_name="s")


G = 4
TCH = EP // K
CH0 = 80
CH1 = (TCH - NS * CH0) // NS
assert CH0 % (2 * G) == 0 and CH1 % (2 * G) == 0 and NS * (CH0 + CH1) == TCH


def _make_agg(with_deg):
    def body(*refs):
        if with_deg:
            (h_hbm, e_hbm, zacc_hbm, ones_hbm, acc_out, deg_out,
             idx0, idx1, rows0, rows1, acc_sh, sg0, sg1, si0, si1) = refs
        else:
            (h_hbm, e_hbm, zacc_hbm, acc_out,
             idx0, idx1, rows0, rows1, acc_sh, sg0, sg1, si0, si1) = refs
        cid = lax.axis_index("c")
        sid = lax.axis_index("s")
        idxb = (idx0, idx1)
        rows = (rows0, rows1)
        sg = (sg0, sg1)
        si = (si0, si1)
        ch_w = lax.select(cid == 0, jnp.int32(CH0), jnp.int32(CH1))
        b0 = cid * (NS * CH0) + sid * ch_w
        ng = ch_w // G
        r0 = sid * RPT

        def wait_gather(b):
            pltpu.make_async_copy(h_hbm.at[pl.ds(0, K)], rows[b],
                                  sg[b]).wait()

        def wait_idx(q):
            pltpu.make_async_copy(e_hbm.at[pl.ds(0, G)], idxb[q],
                                  si[q]).wait()

        def prefetch_idx(gidx, q):
            @pl.when(gidx + 1 < ng)
            def _():
                pltpu.async_copy(e_hbm.at[pl.ds(b0 + (gidx + 1) * G, G)],
                                 idxb[1 - q], si[1 - q])

        pltpu.sync_copy(zacc_hbm.at[pl.ds(r0, RPT)],
                        acc_sh.at[pl.ds(r0, RPT)])
        pltpu.sync_copy(e_hbm.at[pl.ds(b0, G)], idx0)

        if with_deg:
            pltpu.sync_copy(ones_hbm, rows0)
            plsc.subcore_barrier()

            def deg_super(sg_i, c):
                for q in (0, 1):
                    gidx = sg_i * 2 + q

                    @pl.when(gidx > 0)
                    def _():
                        wait_idx(q)
                    prefetch_idx(gidx, q)
                    for p in range(G):
                        pltpu.sync_copy(rows0, acc_sh.at[idxb[q].at[p, 1]],
                                        add=True)
                return c

            lax.fori_loop(0, ng // 2, deg_super, 0)
            plsc.subcore_barrier()
            pltpu.sync_copy(acc_sh.at[pl.ds(r0, RPT)],
                            deg_out.at[cid, pl.ds(r0, RPT)])
            pltpu.sync_copy(zacc_hbm.at[pl.ds(r0, RPT)],
                            acc_sh.at[pl.ds(r0, RPT)])
            pltpu.sync_copy(e_hbm.at[pl.ds(b0, G)], idx0)

        pltpu.async_copy(h_hbm.at[idx0.at[0, 0]], rows0, sg0)
        plsc.subcore_barrier()

        def super_group(sg_i, c):
            for q in (0, 1):
                gidx = sg_i * 2 + q
                prefetch_idx(gidx, q)
                for p in range(G):
                    b = p % 2
                    j = gidx * G + p
                    if p < G - 1:
                        pltpu.async_copy(h_hbm.at[idxb[q].at[p + 1, 0]],
                                         rows[1 - b], sg[1 - b])
                    else:
                        @pl.when(j + 1 < ch_w)
                        def _():
                            wait_idx(1 - q)
                            pltpu.async_copy(h_hbm.at[idxb[1 - q].at[0, 0]],
                                             rows[1 - b], sg[1 - b])
                    wait_gather(b)
                    pltpu.sync_copy(rows[b], acc_sh.at[idxb[q].at[p, 1]],
                                    add=True)
            return c

        lax.fori_loop(0, ng // 2, super_group, 0)
        plsc.subcore_barrier()
        pltpu.sync_copy(acc_sh.at[pl.ds(r0, RPT)],
                        acc_out.at[cid, pl.ds(r0, RPT)])

    out_type = [jax.ShapeDtypeStruct((NC, NP, D), jnp.float32)]
    if with_deg:
        out_type.append(jax.ShapeDtypeStruct((NC, NP, D), jnp.float32))
    return pl.kernel(
        body,
        out_type=out_type,
        mesh=_MESH,
        scratch_types=[
            pltpu.VMEM((G, 2, K), jnp.int32),
            pltpu.VMEM((G, 2, K), jnp.int32),
            pltpu.VMEM((K, D), jnp.float32),
            pltpu.VMEM((K, D), jnp.float32),
            pltpu.VMEM_SHARED((NP, D), jnp.float32),
            pltpu.SemaphoreType.DMA,
            pltpu.SemaphoreType.DMA,
            pltpu.SemaphoreType.DMA,
            pltpu.SemaphoreType.DMA,
        ],
    )


_agg_deg = _make_agg(True)
_agg = _make_agg(False)


_BLK = 1000


def _deg_col(d_ref):
    return d_ref[0, :, :1] + d_ref[1, :, :1]


def _tc1_body(x_ref, s0_ref, s1_ref, d_ref, ws_ref, wn_ref, b_ref, o_ref):
    deg = _deg_col(d_ref)
    hn = (s0_ref[...] + s1_ref[...]) / jnp.maximum(deg, 1.0)
    o_ref[...] = jnp.maximum(
        jnp.dot(x_ref[...], ws_ref[...], preferred_element_type=jnp.float32)
        + jnp.dot(hn, wn_ref[...], preferred_element_type=jnp.float32)
        + b_ref[...], 0.0)


def _tc1(x, s0, s1, dp, ws, wn, b):
    grid = N // _BLK
    row = lambda i: (i, 0)
    full = lambda i: (0, 0)
    return pl.pallas_call(
        _tc1_body,
        grid=(grid,),
        in_specs=[
            pl.BlockSpec((_BLK, D), row),
            pl.BlockSpec((_BLK, D), row),
            pl.BlockSpec((_BLK, D), row),
            pl.BlockSpec((NC, _BLK, D), lambda i: (0, i, 0)),
            pl.BlockSpec((D, D), full),
            pl.BlockSpec((D, D), full),
            pl.BlockSpec((1, D), full),
        ],
        out_specs=pl.BlockSpec((_BLK, D), row),
        out_shape=jax.ShapeDtypeStruct((N, D), jnp.float32),
    )(x, s0, s1, dp, ws, wn, b)


def _tc2_body(h_ref, s0_ref, s1_ref, d_ref, ws_ref, wn_ref, b_ref,
              wc_ref, bc_ref, o_ref, acc_ref):
    i = pl.program_id(0)

    @pl.when(i == 0)
    def _():
        acc_ref[...] = jnp.zeros_like(acc_ref)

    deg = _deg_col(d_ref)
    hn = (s0_ref[...] + s1_ref[...]) / jnp.maximum(deg, 1.0)
    h2 = jnp.maximum(
        jnp.dot(h_ref[...], ws_ref[...], preferred_element_type=jnp.float32)
        + jnp.dot(hn, wn_ref[...], preferred_element_type=jnp.float32)
        + b_ref[...], 0.0)
    acc_ref[...] += jnp.sum(h2, axis=0, keepdims=True)

    @pl.when(i == pl.num_programs(0) - 1)
    def _():
        pooled = acc_ref[...] * (1.0 / N)
        o_ref[...] = (jnp.dot(pooled, wc_ref[...],
                              preferred_element_type=jnp.float32)
                      + bc_ref[...])


def _tc2(h, s0, s1, dp, ws, wn, b, wc, bc):
    grid = N // _BLK
    row = lambda i: (i, 0)
    full = lambda i: (0, 0)
    out = pl.pallas_call(
        _tc2_body,
        grid=(grid,),
        in_specs=[
            pl.BlockSpec((_BLK, D), row),
            pl.BlockSpec((_BLK, D), row),
            pl.BlockSpec((_BLK, D), row),
            pl.BlockSpec((NC, _BLK, D), lambda i: (0, i, 0)),
            pl.BlockSpec((D, D), full),
            pl.BlockSpec((D, D), full),
            pl.BlockSpec((1, D), full),
            pl.BlockSpec((D, C), full),
            pl.BlockSpec((1, C), full),
        ],
        out_specs=pl.BlockSpec((1, C), full),
        out_shape=jax.ShapeDtypeStruct((1, C), jnp.float32),
        scratch_shapes=[pltpu.VMEM((1, D), jnp.float32)],
    )(h, s0, s1, dp, ws, wn, b, wc, bc)
    return out[0]


def kernel(x, edge_index, W_self0, W_neigh0, b0, W_self1, W_neigh1, b1,
           W_cls, b_cls):
    src = edge_index[0]
    dst = edge_index[1]
    pad = EP - src.shape[0]
    src_pad = (jnp.arange(pad, dtype=jnp.int32) * 131) % N
    srcp = jnp.concatenate([src, src_pad])
    dst_pad = N + (jnp.arange(pad, dtype=jnp.int32) % (NP - N))
    dstp = jnp.concatenate([dst, dst_pad])
    e3 = jnp.stack([srcp.reshape(TCH, K), dstp.reshape(TCH, K)], axis=1)

    zacc = jnp.zeros((NP, D), jnp.float32)
    ones128 = jnp.ones((K, D), jnp.float32)

    acc1, degp = _agg_deg(x, e3, zacc, ones128)
    h1 = _tc1(x, acc1[0], acc1[1], degp, W_self0, W_neigh0, b0.reshape(1, D))
    acc2, = _agg(h1, e3, zacc)
    return _tc2(h1, acc2[0], acc2[1], degp, W_self1, W_neigh1,
                b1.reshape(1, D), W_cls, b_cls.reshape(1, C))
